# baseline (speedup 1.0000x reference)
import functools

import jax
import jax.numpy as jnp
from jax import lax
from jax.experimental import pallas as pl
from jax.experimental.pallas import tpu as pltpu
from jax.experimental.pallas import tpu_sc as plsc

_B = 16384
_D = 16
_G = 8
_GW = _G * _D
_NC = 2
_NS = 16
_NW = _NC * _NS
_BPW = _B // _NW
_CHUNK = 128
_NCHUNK = _BPW // _CHUNK

_BLK = 1024


def _gather_body(user_h, pos_h, neg_h, mfu_t, mfi_t, mlu_t, mli_t,
                 o_mfu, o_mfp, o_mfn, o_mlu, o_mlp, o_mln,
                 uidx, pidx, nidx, buf0, buf1,
                 sg0, sg1, so0, so1):
    wid = lax.axis_index("s") * _NC + lax.axis_index("c")
    base = wid * _BPW
    pltpu.sync_copy(user_h.at[pl.ds(base, _BPW)], uidx)
    pltpu.sync_copy(pos_h.at[pl.ds(base, _BPW)], pidx)
    pltpu.sync_copy(neg_h.at[pl.ds(base, _BPW)], nidx)

    units = []
    for tab, idx, dst in ((mfu_t, uidx, o_mfu), (mlu_t, uidx, o_mlu),
                          (mfi_t, pidx, o_mfp), (mli_t, pidx, o_mlp),
                          (mfi_t, nidx, o_mfn), (mli_t, nidx, o_mln)):
        for c in range(_NCHUNK):
            units.append((tab, idx, dst, c))

    bufs = (buf0, buf1)
    gsems = (sg0, sg1)
    osems = (so0, so1)

    def start_gather(u, b):
        tab, idx, _, c = units[u]
        return pltpu.async_copy(
            tab.at[idx.at[pl.ds(c * _CHUNK, _CHUNK)]], bufs[b], gsems[b])

    def start_out(u, b):
        _, _, dst, c = units[u]
        return pltpu.async_copy(
            bufs[b], dst.at[pl.ds(base + c * _CHUNK, _CHUNK)], osems[b])

    n = len(units)
    gath = start_gather(0, 0)
    outc = [None, None]
    for u in range(n):
        b = u % 2
        nb = 1 - b
        gath.wait()
        if u + 1 < n:
            if outc[nb] is not None:
                outc[nb].wait()
                outc[nb] = None
            next_gath = start_gather(u + 1, nb)
        outc[b] = start_out(u, b)
        if u + 1 < n:
            gath = next_gath
    for oc in outc:
        if oc is not None:
            oc.wait()


@jax.jit
def _gather6(user_g, pos_g, neg_g, mfu_t, mfi_t, mlu_t, mli_t):
    mesh = plsc.VectorSubcoreMesh(core_axis_name="c", subcore_axis_name="s")
    out = jax.ShapeDtypeStruct((_B, _GW), jnp.float32)
    f = pl.kernel(
        _gather_body,
        out_type=[out] * 6,
        mesh=mesh,
        scratch_types=(
            [pltpu.VMEM((_BPW,), jnp.int32)] * 3
            + [pltpu.VMEM((_CHUNK, _GW), jnp.float32)] * 2
            + [pltpu.SemaphoreType.DMA] * 4
        ),
    )
    return f(user_g, pos_g, neg_g, mfu_t, mfi_t, mlu_t, mli_t)


def _extract(rows, j):
    acc = None
    for t in range(_G):
        part = jnp.where(j == t, rows[:, t * _D:(t + 1) * _D], 0.0)
        acc = part if acc is None else acc + part
    return acc


def _tower_body(gmfu, gmfp, gmfn, gmlu, gmlp, gmln, ju, jp, jn,
                w1u, w1i, b1, w2, b2, w3, b3, w4, b4, wdm, wdl, bd, out):
    f32 = jnp.float32
    ju_ = ju[...]
    jp_ = jp[...]
    jn_ = jn[...]
    mfu = _extract(gmfu[...], ju_)
    mlu = _extract(gmlu[...], ju_)
    mfp = _extract(gmfp[...], jp_)
    mlpos = _extract(gmlp[...], jp_)
    mfn = _extract(gmfn[...], jn_)
    mlneg = _extract(gmln[...], jn_)
    xu = jnp.dot(mlu, w1u[...], preferred_element_type=f32)
    hp = jnp.maximum(xu + jnp.dot(mlpos, w1i[...],
                                  preferred_element_type=f32) + b1[...], 0.0)
    hn = jnp.maximum(xu + jnp.dot(mlneg, w1i[...],
                                  preferred_element_type=f32) + b1[...], 0.0)
    for w, b in ((w2, b2), (w3, b3), (w4, b4)):
        hp = jnp.maximum(jnp.dot(hp, w[...], preferred_element_type=f32) + b[...], 0.0)
        hn = jnp.maximum(jnp.dot(hn, w[...], preferred_element_type=f32) + b[...], 0.0)
    mfp_v = jax.nn.sigmoid(mfu * mfp)
    mfn_v = jax.nn.sigmoid(mfu * mfn)
    sp = (jnp.dot(mfp_v, wdm[...], preferred_element_type=f32)
          + jnp.dot(hp, wdl[...], preferred_element_type=f32) + bd[0, 0])
    sn = (jnp.dot(mfn_v, wdm[...], preferred_element_type=f32)
          + jnp.dot(hn, wdl[...], preferred_element_type=f32) + bd[0, 0])
    out[...] = jnp.concatenate([sp, sn], axis=1)


def _tower(gmfu, gmfp, gmfn, gmlu, gmlp, gmln, ju, jp, jn,
           w1u, w1i, b1, w2, b2, w3, b3, w4, b4, wdm, wdl, bd,
           interpret=False):
    gspec = pl.BlockSpec((_BLK, _GW), lambda i: (i, 0))
    jspec = pl.BlockSpec((_BLK, 1), lambda i: (i, 0))

    def _full(a):
        return pl.BlockSpec(a.shape, lambda i: (0,) * a.ndim)

    weights = (w1u, w1i, b1, w2, b2, w3, b3, w4, b4, wdm, wdl, bd)
    return pl.pallas_call(
        _tower_body,
        grid=(_B // _BLK,),
        in_specs=[gspec] * 6 + [jspec] * 3 + [_full(w) for w in weights],
        out_specs=pl.BlockSpec((_BLK, 2), lambda i: (i, 0)),
        out_shape=jax.ShapeDtypeStruct((_B, 2), jnp.float32),
        interpret=interpret,
    )(gmfu, gmfp, gmfn, gmlu, gmlp, gmln, ju, jp, jn, *weights)


def kernel(user, pos_item, neg_item,
           mf_user_table, mf_item_table, mlp_user_table, mlp_item_table,
           W1, b1, W2, b2, W3, b3, W4, b4, Wd, bd):
    user = user.astype(jnp.int32)
    pos = pos_item.astype(jnp.int32)
    neg = neg_item.reshape(-1).astype(jnp.int32)
    gmfu, gmfp, gmfn, gmlu, gmlp, gmln = _gather6(
        user // _G, pos // _G, neg // _G,
        mf_user_table.reshape(-1, _GW), mf_item_table.reshape(-1, _GW),
        mlp_user_table.reshape(-1, _GW), mlp_item_table.reshape(-1, _GW))
    logits = _tower(
        gmfu, gmfp, gmfn, gmlu, gmlp, gmln,
        (user % _G).reshape(-1, 1), (pos % _G).reshape(-1, 1),
        (neg % _G).reshape(-1, 1),
        W1[:_D], W1[_D:], b1.reshape(1, -1),
        W2, b2.reshape(1, -1), W3, b3.reshape(1, -1), W4, b4.reshape(1, -1),
        Wd[:_D], Wd[_D:], bd.reshape(1, 1))
    return logits


# speedup vs baseline: 1.1038x; 1.1038x over previous
import functools

import jax
import jax.numpy as jnp
from jax import lax
from jax.experimental import pallas as pl
from jax.experimental.pallas import tpu as pltpu
from jax.experimental.pallas import tpu_sc as plsc

_B = 16384
_D = 16
_G = 8
_GW = _G * _D
_NC = 2
_NS = 16
_NW = _NC * _NS
_BPW = _B // _NW
_CHUNK = 128
_NCHUNK = _BPW // _CHUNK

_BLK = 1024


def _gather_body(user_h, pos_h, neg_h, mfu_t, mfi_t, mlu_t, mli_t,
                 o_mfu, o_mfp, o_mfn, o_mlu, o_mlp, o_mln,
                 uidx, pidx, nidx, buf0, buf1,
                 sg0, sg1, so0, so1):
    wid = lax.axis_index("s") * _NC + lax.axis_index("c")
    base = wid * _BPW
    pltpu.sync_copy(user_h.at[pl.ds(base, _BPW)], uidx)
    pltpu.sync_copy(pos_h.at[pl.ds(base, _BPW)], pidx)
    pltpu.sync_copy(neg_h.at[pl.ds(base, _BPW)], nidx)

    units = []
    for tab, idx, dst in ((mfu_t, uidx, o_mfu), (mlu_t, uidx, o_mlu),
                          (mfi_t, pidx, o_mfp), (mli_t, pidx, o_mlp),
                          (mfi_t, nidx, o_mfn), (mli_t, nidx, o_mln)):
        for c in range(_NCHUNK):
            units.append((tab, idx, dst, c))

    bufs = (buf0, buf1)
    gsems = (sg0, sg1)
    osems = (so0, so1)

    def start_gather(u, b):
        tab, idx, _, c = units[u]
        return pltpu.async_copy(
            tab.at[idx.at[pl.ds(c * _CHUNK, _CHUNK)]], bufs[b], gsems[b])

    def start_out(u, b):
        _, _, dst, c = units[u]
        return pltpu.async_copy(
            bufs[b], dst.at[pl.ds(base + c * _CHUNK, _CHUNK)], osems[b])

    n = len(units)
    gath = start_gather(0, 0)
    outc = [None, None]
    for u in range(n):
        b = u % 2
        nb = 1 - b
        gath.wait()
        if u + 1 < n:
            if outc[nb] is not None:
                outc[nb].wait()
                outc[nb] = None
            next_gath = start_gather(u + 1, nb)
        outc[b] = start_out(u, b)
        if u + 1 < n:
            gath = next_gath
    for oc in outc:
        if oc is not None:
            oc.wait()


@jax.jit
def _gather6(user_g, pos_g, neg_g, mfu_t, mfi_t, mlu_t, mli_t):
    mesh = plsc.VectorSubcoreMesh(core_axis_name="c", subcore_axis_name="s")
    out = jax.ShapeDtypeStruct((_B, _GW), jnp.float32)
    f = pl.kernel(
        _gather_body,
        out_type=[out] * 6,
        mesh=mesh,
        scratch_types=(
            [pltpu.VMEM((_BPW,), jnp.int32)] * 3
            + [pltpu.VMEM((_CHUNK, _GW), jnp.float32)] * 2
            + [pltpu.SemaphoreType.DMA] * 4
        ),
    )
    return f(user_g, pos_g, neg_g, mfu_t, mfi_t, mlu_t, mli_t)


_U = 1000000
_NG = 125440
_RB = 1280
_TAIL = (_G - 1) * _NG


def _regroup_body(x1, x2, x3, x4, e1, e2, e3, e4, y1, y2, y3, y4, slab, sem):
    i = pl.program_id(0)
    xs = (x1, x2, x3, x4)
    es = (e1, e2, e3, e4)

    def src(ti, j):
        if j < _G - 1:
            return xs[ti].at[:, pl.ds(j * _NG + i * _RB, _RB)]
        return es[ti].at[:, pl.ds(i * _RB, _RB)]

    for ti in range(4):
        for j in range(_G):
            pltpu.make_async_copy(src(ti, j), slab.at[ti * _G + j], sem).start()
    for ti in range(4):
        for j in range(_G):
            pltpu.make_async_copy(src(ti, j), slab.at[ti * _G + j], sem).wait()
    ys = (y1, y2, y3, y4)
    for ti in range(4):
        for j in range(_G):
            ys[ti][:, j * _D:(j + 1) * _D] = jnp.transpose(
                slab[ti * _G + j])


def _regroup(t1, t2, t3, t4):
    nblk = _NG // _RB
    in_spec = pl.BlockSpec(memory_space=pl.ANY)
    out_spec = pl.BlockSpec((_RB, _GW), lambda i: (i, 0))
    out = jax.ShapeDtypeStruct((_NG, _GW), jnp.float32)
    tails = [jnp.pad(t[:, _TAIL:], ((0, 0), (0, _NG - (_U - _TAIL))))
             for t in (t1, t2, t3, t4)]
    return pl.pallas_call(
        _regroup_body,
        grid=(nblk,),
        in_specs=[in_spec] * 8,
        out_specs=[out_spec] * 4,
        out_shape=[out] * 4,
        scratch_shapes=[
            pltpu.VMEM((4 * _G, _D, _RB), jnp.float32),
            pltpu.SemaphoreType.DMA,
        ],
    )(t1, t2, t3, t4, *tails)


def _extract(rows, j):
    acc = None
    for t in range(_G):
        part = jnp.where(j == t, rows[:, t * _D:(t + 1) * _D], 0.0)
        acc = part if acc is None else acc + part
    return acc


def _tower_body(gmfu, gmfp, gmfn, gmlu, gmlp, gmln, ju, jp, jn,
                w1u, w1i, b1, w2, b2, w3, b3, w4, b4, wdm, wdl, bd, out):
    f32 = jnp.float32
    ju_ = ju[...]
    jp_ = jp[...]
    jn_ = jn[...]
    mfu = _extract(gmfu[...], ju_)
    mlu = _extract(gmlu[...], ju_)
    mfp = _extract(gmfp[...], jp_)
    mlpos = _extract(gmlp[...], jp_)
    mfn = _extract(gmfn[...], jn_)
    mlneg = _extract(gmln[...], jn_)
    xu = jnp.dot(mlu, w1u[...], preferred_element_type=f32)
    hp = jnp.maximum(xu + jnp.dot(mlpos, w1i[...],
                                  preferred_element_type=f32) + b1[...], 0.0)
    hn = jnp.maximum(xu + jnp.dot(mlneg, w1i[...],
                                  preferred_element_type=f32) + b1[...], 0.0)
    for w, b in ((w2, b2), (w3, b3), (w4, b4)):
        hp = jnp.maximum(jnp.dot(hp, w[...], preferred_element_type=f32) + b[...], 0.0)
        hn = jnp.maximum(jnp.dot(hn, w[...], preferred_element_type=f32) + b[...], 0.0)
    mfp_v = jax.nn.sigmoid(mfu * mfp)
    mfn_v = jax.nn.sigmoid(mfu * mfn)
    sp = (jnp.dot(mfp_v, wdm[...], preferred_element_type=f32)
          + jnp.dot(hp, wdl[...], preferred_element_type=f32) + bd[0, 0])
    sn = (jnp.dot(mfn_v, wdm[...], preferred_element_type=f32)
          + jnp.dot(hn, wdl[...], preferred_element_type=f32) + bd[0, 0])
    out[...] = jnp.concatenate([sp, sn], axis=1)


def _tower(gmfu, gmfp, gmfn, gmlu, gmlp, gmln, ju, jp, jn,
           w1u, w1i, b1, w2, b2, w3, b3, w4, b4, wdm, wdl, bd,
           interpret=False):
    gspec = pl.BlockSpec((_BLK, _GW), lambda i: (i, 0))
    jspec = pl.BlockSpec((_BLK, 1), lambda i: (i, 0))

    def _full(a):
        return pl.BlockSpec(a.shape, lambda i: (0,) * a.ndim)

    weights = (w1u, w1i, b1, w2, b2, w3, b3, w4, b4, wdm, wdl, bd)
    return pl.pallas_call(
        _tower_body,
        grid=(_B // _BLK,),
        in_specs=[gspec] * 6 + [jspec] * 3 + [_full(w) for w in weights],
        out_specs=pl.BlockSpec((_BLK, 2), lambda i: (i, 0)),
        out_shape=jax.ShapeDtypeStruct((_B, 2), jnp.float32),
        interpret=interpret,
    )(gmfu, gmfp, gmfn, gmlu, gmlp, gmln, ju, jp, jn, *weights)


def kernel(user, pos_item, neg_item,
           mf_user_table, mf_item_table, mlp_user_table, mlp_item_table,
           W1, b1, W2, b2, W3, b3, W4, b4, Wd, bd):
    user = user.astype(jnp.int32)
    pos = pos_item.astype(jnp.int32)
    neg = neg_item.reshape(-1).astype(jnp.int32)
    g_mfu_t, g_mfi_t, g_mlu_t, g_mli_t = _regroup(
        mf_user_table.T, mf_item_table.T,
        mlp_user_table.T, mlp_item_table.T)
    gmfu, gmfp, gmfn, gmlu, gmlp, gmln = _gather6(
        user % _NG, pos % _NG, neg % _NG,
        g_mfu_t, g_mfi_t, g_mlu_t, g_mli_t)
    logits = _tower(
        gmfu, gmfp, gmfn, gmlu, gmlp, gmln,
        (user // _NG).astype(jnp.int32).reshape(-1, 1),
        (pos // _NG).astype(jnp.int32).reshape(-1, 1),
        (neg // _NG).astype(jnp.int32).reshape(-1, 1),
        W1[:_D], W1[_D:], b1.reshape(1, -1),
        W2, b2.reshape(1, -1), W3, b3.reshape(1, -1), W4, b4.reshape(1, -1),
        Wd[:_D], Wd[_D:], bd.reshape(1, 1))
    return logits


# speedup vs baseline: 2.2759x; 2.0618x over previous
import functools

import jax
import jax.numpy as jnp
from jax import lax
from jax.experimental import pallas as pl
from jax.experimental.pallas import tpu as pltpu
from jax.experimental.pallas import tpu_sc as plsc

_B = 16384
_D = 16
_G = 8
_GW = _G * _D
_NC = 2
_NS = 16
_NW = _NC * _NS
_BPW = _B // _NW
_CHUNK = 128
_NCHUNK = _BPW // _CHUNK

_BLK = 1024


def _gather_body(user_h, pos_h, neg_h, mfu_t, mfi_t, mlu_t, mli_t,
                 o_mfu, o_mfp, o_mfn, o_mlu, o_mlp, o_mln,
                 uidx, pidx, nidx, buf0, buf1,
                 sg0, sg1, so0, so1):
    wid = lax.axis_index("s") * _NC + lax.axis_index("c")
    base = wid * _BPW
    pltpu.sync_copy(user_h.at[pl.ds(base, _BPW)], uidx)
    pltpu.sync_copy(pos_h.at[pl.ds(base, _BPW)], pidx)
    pltpu.sync_copy(neg_h.at[pl.ds(base, _BPW)], nidx)

    units = []
    for tab, idx, dst in ((mfu_t, uidx, o_mfu), (mlu_t, uidx, o_mlu),
                          (mfi_t, pidx, o_mfp), (mli_t, pidx, o_mlp),
                          (mfi_t, nidx, o_mfn), (mli_t, nidx, o_mln)):
        for c in range(_NCHUNK):
            units.append((tab, idx, dst, c))

    bufs = (buf0, buf1)
    gsems = (sg0, sg1)
    osems = (so0, so1)

    def start_gather(u, b):
        tab, idx, _, c = units[u]
        return pltpu.async_copy(
            tab.at[idx.at[pl.ds(c * _CHUNK, _CHUNK)]], bufs[b], gsems[b])

    def start_out(u, b):
        _, _, dst, c = units[u]
        return pltpu.async_copy(
            bufs[b], dst.at[pl.ds(base + c * _CHUNK, _CHUNK)], osems[b])

    n = len(units)
    gath = start_gather(0, 0)
    outc = [None, None]
    for u in range(n):
        b = u % 2
        nb = 1 - b
        gath.wait()
        if u + 1 < n:
            if outc[nb] is not None:
                outc[nb].wait()
                outc[nb] = None
            next_gath = start_gather(u + 1, nb)
        outc[b] = start_out(u, b)
        if u + 1 < n:
            gath = next_gath
    for oc in outc:
        if oc is not None:
            oc.wait()


@jax.jit
def _gather6(user_g, pos_g, neg_g, mfu_t, mfi_t, mlu_t, mli_t):
    mesh = plsc.VectorSubcoreMesh(core_axis_name="c", subcore_axis_name="s")
    out = jax.ShapeDtypeStruct((_B, _GW), jnp.float32)
    f = pl.kernel(
        _gather_body,
        out_type=[out] * 6,
        mesh=mesh,
        scratch_types=(
            [pltpu.VMEM((_BPW,), jnp.int32)] * 3
            + [pltpu.VMEM((_CHUNK, _GW), jnp.float32)] * 2
            + [pltpu.SemaphoreType.DMA] * 4
        ),
    )
    return f(user_g, pos_g, neg_g, mfu_t, mfi_t, mlu_t, mli_t)


_U = 1000000
_NG = 125440
_RB = 2560
_TAIL = (_G - 1) * _NG


def _regroup_body(x1, x2, x3, x4, e1, e2, e3, e4, y1, y2, y3, y4, slab, sem):
    i = pl.program_id(0)
    nblk = pl.num_programs(0)
    xs = (x1, x2, x3, x4)
    es = (e1, e2, e3, e4)

    def src(ti, j, step):
        if j < _G - 1:
            return xs[ti].at[:, pl.ds(j * _NG + step * _RB, _RB)]
        return es[ti].at[:, pl.ds(step * _RB, _RB)]

    def start_all(step, buf):
        for ti in range(4):
            for j in range(_G):
                pltpu.make_async_copy(
                    src(ti, j, step),
                    slab.at[buf, ti, pl.ds(j * _D, _D)], sem.at[buf]).start()

    @pl.when(i == 0)
    def _():
        start_all(0, 0)

    @pl.when(i + 1 < nblk)
    def _():
        start_all(i + 1, (i + 1) % 2)

    eye = jnp.eye(_GW, dtype=jnp.float32)
    ys = (y1, y2, y3, y4)
    b = i % 2
    for ti in range(4):
        for j in range(_G):
            pltpu.make_async_copy(
                src(ti, j, i), slab.at[b, ti, pl.ds(j * _D, _D)], sem.at[b]).wait()
        ys[ti][...] = jax.lax.dot_general(
            slab[b, ti], eye, (((0,), (0,)), ((), ())),
            precision=jax.lax.Precision.HIGHEST,
            preferred_element_type=jnp.float32)


def _regroup(t1, t2, t3, t4):
    nblk = _NG // _RB
    in_spec = pl.BlockSpec(memory_space=pl.ANY)
    out_spec = pl.BlockSpec((_RB, _GW), lambda i: (i, 0))
    out = jax.ShapeDtypeStruct((_NG, _GW), jnp.float32)
    tails = [jnp.pad(t[:, _TAIL:], ((0, 0), (0, _NG - (_U - _TAIL))))
             for t in (t1, t2, t3, t4)]
    return pl.pallas_call(
        _regroup_body,
        grid=(nblk,),
        in_specs=[in_spec] * 8,
        out_specs=[out_spec] * 4,
        out_shape=[out] * 4,
        scratch_shapes=[
            pltpu.VMEM((2, 4, _GW, _RB), jnp.float32),
            pltpu.SemaphoreType.DMA((2,)),
        ],
    )(t1, t2, t3, t4, *tails)


def _extract(rows, j):
    acc = None
    for t in range(_G):
        part = jnp.where(j == t, rows[:, t * _D:(t + 1) * _D], 0.0)
        acc = part if acc is None else acc + part
    return acc


def _tower_body(gmfu, gmfp, gmfn, gmlu, gmlp, gmln, ju, jp, jn,
                w1u, w1i, b1, w2, b2, w3, b3, w4, b4, wdm, wdl, bd, out):
    f32 = jnp.float32
    ju_ = ju[...]
    jp_ = jp[...]
    jn_ = jn[...]
    mfu = _extract(gmfu[...], ju_)
    mlu = _extract(gmlu[...], ju_)
    mfp = _extract(gmfp[...], jp_)
    mlpos = _extract(gmlp[...], jp_)
    mfn = _extract(gmfn[...], jn_)
    mlneg = _extract(gmln[...], jn_)
    xu = jnp.dot(mlu, w1u[...], preferred_element_type=f32)
    hp = jnp.maximum(xu + jnp.dot(mlpos, w1i[...],
                                  preferred_element_type=f32) + b1[...], 0.0)
    hn = jnp.maximum(xu + jnp.dot(mlneg, w1i[...],
                                  preferred_element_type=f32) + b1[...], 0.0)
    for w, b in ((w2, b2), (w3, b3), (w4, b4)):
        hp = jnp.maximum(jnp.dot(hp, w[...], preferred_element_type=f32) + b[...], 0.0)
        hn = jnp.maximum(jnp.dot(hn, w[...], preferred_element_type=f32) + b[...], 0.0)
    mfp_v = jax.nn.sigmoid(mfu * mfp)
    mfn_v = jax.nn.sigmoid(mfu * mfn)
    sp = (jnp.dot(mfp_v, wdm[...], preferred_element_type=f32)
          + jnp.dot(hp, wdl[...], preferred_element_type=f32) + bd[0, 0])
    sn = (jnp.dot(mfn_v, wdm[...], preferred_element_type=f32)
          + jnp.dot(hn, wdl[...], preferred_element_type=f32) + bd[0, 0])
    out[...] = jnp.concatenate([sp, sn], axis=1)


def _tower(gmfu, gmfp, gmfn, gmlu, gmlp, gmln, ju, jp, jn,
           w1u, w1i, b1, w2, b2, w3, b3, w4, b4, wdm, wdl, bd,
           interpret=False):
    gspec = pl.BlockSpec((_BLK, _GW), lambda i: (i, 0))
    jspec = pl.BlockSpec((_BLK, 1), lambda i: (i, 0))

    def _full(a):
        return pl.BlockSpec(a.shape, lambda i: (0,) * a.ndim)

    weights = (w1u, w1i, b1, w2, b2, w3, b3, w4, b4, wdm, wdl, bd)
    return pl.pallas_call(
        _tower_body,
        grid=(_B // _BLK,),
        in_specs=[gspec] * 6 + [jspec] * 3 + [_full(w) for w in weights],
        out_specs=pl.BlockSpec((_BLK, 2), lambda i: (i, 0)),
        out_shape=jax.ShapeDtypeStruct((_B, 2), jnp.float32),
        interpret=interpret,
    )(gmfu, gmfp, gmfn, gmlu, gmlp, gmln, ju, jp, jn, *weights)


def kernel(user, pos_item, neg_item,
           mf_user_table, mf_item_table, mlp_user_table, mlp_item_table,
           W1, b1, W2, b2, W3, b3, W4, b4, Wd, bd):
    user = user.astype(jnp.int32)
    pos = pos_item.astype(jnp.int32)
    neg = neg_item.reshape(-1).astype(jnp.int32)
    g_mfu_t, g_mfi_t, g_mlu_t, g_mli_t = _regroup(
        mf_user_table.T, mf_item_table.T,
        mlp_user_table.T, mlp_item_table.T)
    gmfu, gmfp, gmfn, gmlu, gmlp, gmln = _gather6(
        user % _NG, pos % _NG, neg % _NG,
        g_mfu_t, g_mfi_t, g_mlu_t, g_mli_t)
    logits = _tower(
        gmfu, gmfp, gmfn, gmlu, gmlp, gmln,
        (user // _NG).astype(jnp.int32).reshape(-1, 1),
        (pos // _NG).astype(jnp.int32).reshape(-1, 1),
        (neg // _NG).astype(jnp.int32).reshape(-1, 1),
        W1[:_D], W1[_D:], b1.reshape(1, -1),
        W2, b2.reshape(1, -1), W3, b3.reshape(1, -1), W4, b4.reshape(1, -1),
        Wd[:_D], Wd[_D:], bd.reshape(1, 1))
    return logits


# speedup vs baseline: 3.5424x; 1.5565x over previous
import functools

import jax
import jax.numpy as jnp
from jax import lax
from jax.experimental import pallas as pl
from jax.experimental.pallas import tpu as pltpu
from jax.experimental.pallas import tpu_sc as plsc

_B = 16384
_D = 16
_G = 8
_GW = _G * _D
_NC = 2
_NS = 16
_NW = _NC * _NS
_BPW = _B // _NW
_CHUNK = 128
_NCHUNK = _BPW // _CHUNK

_BLK = 1024


def _gather_body(user_h, pos_h, neg_h, mfu_t, mfi_t, mlu_t, mli_t,
                 o_mfu, o_mfp, o_mfn, o_mlu, o_mlp, o_mln,
                 uidx, pidx, nidx, buf0, buf1,
                 sg0, sg1, so0, so1):
    wid = lax.axis_index("s") * _NC + lax.axis_index("c")
    base = wid * _BPW
    pltpu.sync_copy(user_h.at[pl.ds(base, _BPW)], uidx)
    pltpu.sync_copy(pos_h.at[pl.ds(base, _BPW)], pidx)
    pltpu.sync_copy(neg_h.at[pl.ds(base, _BPW)], nidx)

    units = []
    for tab, idx, dst in ((mfu_t, uidx, o_mfu), (mlu_t, uidx, o_mlu),
                          (mfi_t, pidx, o_mfp), (mli_t, pidx, o_mlp),
                          (mfi_t, nidx, o_mfn), (mli_t, nidx, o_mln)):
        for c in range(_NCHUNK):
            units.append((tab, idx, dst, c))

    bufs = (buf0, buf1)
    gsems = (sg0, sg1)
    osems = (so0, so1)

    def start_gather(u, b):
        tab, idx, _, c = units[u]
        return pltpu.async_copy(
            tab.at[idx.at[pl.ds(c * _CHUNK, _CHUNK)]], bufs[b], gsems[b])

    def start_out(u, b):
        _, _, dst, c = units[u]
        return pltpu.async_copy(
            bufs[b], dst.at[pl.ds(base + c * _CHUNK, _CHUNK)], osems[b])

    n = len(units)
    gath = start_gather(0, 0)
    outc = [None, None]
    for u in range(n):
        b = u % 2
        nb = 1 - b
        gath.wait()
        if u + 1 < n:
            if outc[nb] is not None:
                outc[nb].wait()
                outc[nb] = None
            next_gath = start_gather(u + 1, nb)
        outc[b] = start_out(u, b)
        if u + 1 < n:
            gath = next_gath
    for oc in outc:
        if oc is not None:
            oc.wait()


@jax.jit
def _gather6(user_g, pos_g, neg_g, mfu_t, mfi_t, mlu_t, mli_t):
    mesh = plsc.VectorSubcoreMesh(core_axis_name="c", subcore_axis_name="s")
    out = jax.ShapeDtypeStruct((_B, _GW), jnp.float32)
    f = pl.kernel(
        _gather_body,
        out_type=[out] * 6,
        mesh=mesh,
        scratch_types=(
            [pltpu.VMEM((_BPW,), jnp.int32)] * 3
            + [pltpu.VMEM((_CHUNK, _GW), jnp.float32)] * 2
            + [pltpu.SemaphoreType.DMA] * 4
        ),
    )
    return f(user_g, pos_g, neg_g, mfu_t, mfi_t, mlu_t, mli_t)


_U = 1000000
_NG = 125440
_RB = 2560
_TAIL = (_G - 1) * _NG


def _regroup_body(x1, x2, x3, x4, e1, e2, e3, e4, y1, y2, y3, y4, slab, sem):
    i = pl.program_id(0)
    nblk = pl.num_programs(0)
    xs = (x1, x2, x3, x4)
    es = (e1, e2, e3, e4)

    def src(ti, j, step):
        if j < _G - 1:
            return xs[ti].at[:, pl.ds(j * _NG + step * _RB, _RB)]
        return es[ti].at[:, pl.ds(step * _RB, _RB)]

    def start_all(step, buf):
        for ti in range(4):
            for j in range(_G):
                pltpu.make_async_copy(
                    src(ti, j, step),
                    slab.at[buf, ti, pl.ds(j * _D, _D)], sem.at[buf]).start()

    @pl.when(i == 0)
    def _():
        start_all(0, 0)

    @pl.when(i + 1 < nblk)
    def _():
        start_all(i + 1, (i + 1) % 2)

    eye = jnp.eye(_GW, dtype=jnp.bfloat16)
    ys = (y1, y2, y3, y4)
    b = i % 2
    dims = (((0,), (0,)), ((), ()))
    for ti in range(4):
        for j in range(_G):
            pltpu.make_async_copy(
                src(ti, j, i), slab.at[b, ti, pl.ds(j * _D, _D)], sem.at[b]).wait()
        x = slab[b, ti]
        hi = x.astype(jnp.bfloat16)
        lo = (x - hi.astype(jnp.float32)).astype(jnp.bfloat16)
        yh = jax.lax.dot_general(hi, eye, dims,
                                 preferred_element_type=jnp.float32)
        yl = jax.lax.dot_general(lo, eye, dims,
                                 preferred_element_type=jnp.float32)
        ys[ti][...] = yh + yl


def _regroup(t1, t2, t3, t4):
    nblk = _NG // _RB
    in_spec = pl.BlockSpec(memory_space=pl.ANY)
    out_spec = pl.BlockSpec((_RB, _GW), lambda i: (i, 0))
    out = jax.ShapeDtypeStruct((_NG, _GW), jnp.float32)
    tails = [jnp.pad(t[:, _TAIL:], ((0, 0), (0, _NG - (_U - _TAIL))))
             for t in (t1, t2, t3, t4)]
    return pl.pallas_call(
        _regroup_body,
        grid=(nblk,),
        in_specs=[in_spec] * 8,
        out_specs=[out_spec] * 4,
        out_shape=[out] * 4,
        scratch_shapes=[
            pltpu.VMEM((2, 4, _GW, _RB), jnp.float32),
            pltpu.SemaphoreType.DMA((2,)),
        ],
    )(t1, t2, t3, t4, *tails)


def _extract(rows, j):
    acc = None
    for t in range(_G):
        part = jnp.where(j == t, rows[:, t * _D:(t + 1) * _D], 0.0)
        acc = part if acc is None else acc + part
    return acc


def _tower_body(gmfu, gmfp, gmfn, gmlu, gmlp, gmln, ju, jp, jn,
                w1u, w1i, b1, w2, b2, w3, b3, w4, b4, wdm, wdl, bd, out):
    f32 = jnp.float32
    ju_ = ju[...]
    jp_ = jp[...]
    jn_ = jn[...]
    mfu = _extract(gmfu[...], ju_)
    mlu = _extract(gmlu[...], ju_)
    mfp = _extract(gmfp[...], jp_)
    mlpos = _extract(gmlp[...], jp_)
    mfn = _extract(gmfn[...], jn_)
    mlneg = _extract(gmln[...], jn_)
    xu = jnp.dot(mlu, w1u[...], preferred_element_type=f32)
    hp = jnp.maximum(xu + jnp.dot(mlpos, w1i[...],
                                  preferred_element_type=f32) + b1[...], 0.0)
    hn = jnp.maximum(xu + jnp.dot(mlneg, w1i[...],
                                  preferred_element_type=f32) + b1[...], 0.0)
    for w, b in ((w2, b2), (w3, b3), (w4, b4)):
        hp = jnp.maximum(jnp.dot(hp, w[...], preferred_element_type=f32) + b[...], 0.0)
        hn = jnp.maximum(jnp.dot(hn, w[...], preferred_element_type=f32) + b[...], 0.0)
    mfp_v = jax.nn.sigmoid(mfu * mfp)
    mfn_v = jax.nn.sigmoid(mfu * mfn)
    sp = (jnp.dot(mfp_v, wdm[...], preferred_element_type=f32)
          + jnp.dot(hp, wdl[...], preferred_element_type=f32) + bd[0, 0])
    sn = (jnp.dot(mfn_v, wdm[...], preferred_element_type=f32)
          + jnp.dot(hn, wdl[...], preferred_element_type=f32) + bd[0, 0])
    out[...] = jnp.concatenate([sp, sn], axis=1)


def _tower(gmfu, gmfp, gmfn, gmlu, gmlp, gmln, ju, jp, jn,
           w1u, w1i, b1, w2, b2, w3, b3, w4, b4, wdm, wdl, bd,
           interpret=False):
    gspec = pl.BlockSpec((_BLK, _GW), lambda i: (i, 0))
    jspec = pl.BlockSpec((_BLK, 1), lambda i: (i, 0))

    def _full(a):
        return pl.BlockSpec(a.shape, lambda i: (0,) * a.ndim)

    weights = (w1u, w1i, b1, w2, b2, w3, b3, w4, b4, wdm, wdl, bd)
    return pl.pallas_call(
        _tower_body,
        grid=(_B // _BLK,),
        in_specs=[gspec] * 6 + [jspec] * 3 + [_full(w) for w in weights],
        out_specs=pl.BlockSpec((_BLK, 2), lambda i: (i, 0)),
        out_shape=jax.ShapeDtypeStruct((_B, 2), jnp.float32),
        interpret=interpret,
    )(gmfu, gmfp, gmfn, gmlu, gmlp, gmln, ju, jp, jn, *weights)


def kernel(user, pos_item, neg_item,
           mf_user_table, mf_item_table, mlp_user_table, mlp_item_table,
           W1, b1, W2, b2, W3, b3, W4, b4, Wd, bd):
    user = user.astype(jnp.int32)
    pos = pos_item.astype(jnp.int32)
    neg = neg_item.reshape(-1).astype(jnp.int32)
    g_mfu_t, g_mfi_t, g_mlu_t, g_mli_t = _regroup(
        mf_user_table.T, mf_item_table.T,
        mlp_user_table.T, mlp_item_table.T)
    gmfu, gmfp, gmfn, gmlu, gmlp, gmln = _gather6(
        user % _NG, pos % _NG, neg % _NG,
        g_mfu_t, g_mfi_t, g_mlu_t, g_mli_t)
    logits = _tower(
        gmfu, gmfp, gmfn, gmlu, gmlp, gmln,
        (user // _NG).astype(jnp.int32).reshape(-1, 1),
        (pos // _NG).astype(jnp.int32).reshape(-1, 1),
        (neg // _NG).astype(jnp.int32).reshape(-1, 1),
        W1[:_D], W1[_D:], b1.reshape(1, -1),
        W2, b2.reshape(1, -1), W3, b3.reshape(1, -1), W4, b4.reshape(1, -1),
        Wd[:_D], Wd[_D:], bd.reshape(1, 1))
    return logits


# speedup vs baseline: 3.6652x; 1.0347x over previous
import functools

import jax
import jax.numpy as jnp
from jax import lax
from jax.experimental import pallas as pl
from jax.experimental.pallas import tpu as pltpu
from jax.experimental.pallas import tpu_sc as plsc

_B = 16384
_D = 16
_G = 8
_GW = _G * _D
_NC = 2
_NS = 16
_NW = _NC * _NS
_BPW = _B // _NW
_CHUNK = 128
_NCHUNK = _BPW // _CHUNK

_BLK = 1024


def _gather_body(user_h, pos_h, neg_h, mfu_t, mfi_t, mlu_t, mli_t,
                 o_mfu, o_mfp, o_mfn, o_mlu, o_mlp, o_mln,
                 uidx, pidx, nidx, buf0, buf1,
                 sg0, sg1, so0, so1):
    wid = lax.axis_index("s") * _NC + lax.axis_index("c")
    base = wid * _BPW
    pltpu.sync_copy(user_h.at[pl.ds(base, _BPW)], uidx)
    pltpu.sync_copy(pos_h.at[pl.ds(base, _BPW)], pidx)
    pltpu.sync_copy(neg_h.at[pl.ds(base, _BPW)], nidx)

    units = []
    for tab, idx, dst in ((mfu_t, uidx, o_mfu), (mlu_t, uidx, o_mlu),
                          (mfi_t, pidx, o_mfp), (mli_t, pidx, o_mlp),
                          (mfi_t, nidx, o_mfn), (mli_t, nidx, o_mln)):
        for c in range(_NCHUNK):
            units.append((tab, idx, dst, c))

    bufs = (buf0, buf1)
    gsems = (sg0, sg1)
    osems = (so0, so1)

    def start_gather(u, b):
        tab, idx, _, c = units[u]
        return pltpu.async_copy(
            tab.at[idx.at[pl.ds(c * _CHUNK, _CHUNK)]], bufs[b], gsems[b])

    def start_out(u, b):
        _, _, dst, c = units[u]
        return pltpu.async_copy(
            bufs[b], dst.at[pl.ds(base + c * _CHUNK, _CHUNK)], osems[b])

    n = len(units)
    gath = start_gather(0, 0)
    outc = [None, None]
    for u in range(n):
        b = u % 2
        nb = 1 - b
        gath.wait()
        if u + 1 < n:
            if outc[nb] is not None:
                outc[nb].wait()
                outc[nb] = None
            next_gath = start_gather(u + 1, nb)
        outc[b] = start_out(u, b)
        if u + 1 < n:
            gath = next_gath
    for oc in outc:
        if oc is not None:
            oc.wait()


@jax.jit
def _gather6(user_g, pos_g, neg_g, mfu_t, mfi_t, mlu_t, mli_t):
    mesh = plsc.VectorSubcoreMesh(core_axis_name="c", subcore_axis_name="s")
    out = jax.ShapeDtypeStruct((_B, _GW), jnp.float32)
    f = pl.kernel(
        _gather_body,
        out_type=[out] * 6,
        mesh=mesh,
        scratch_types=(
            [pltpu.VMEM((_BPW,), jnp.int32)] * 3
            + [pltpu.VMEM((_CHUNK, _GW), jnp.float32)] * 2
            + [pltpu.SemaphoreType.DMA] * 4
        ),
    )
    return f(user_g, pos_g, neg_g, mfu_t, mfi_t, mlu_t, mli_t)


_U = 1000000
_NG = 125440
_RB = 4480
_TAIL = (_G - 1) * _NG


def _regroup_body(x1, x2, x3, x4, e1, e2, e3, e4, y1, y2, y3, y4, slab, sem):
    i = pl.program_id(0)
    nblk = pl.num_programs(0)
    xs = (x1, x2, x3, x4)
    es = (e1, e2, e3, e4)

    def src(ti, j, step):
        if j < _G - 1:
            return xs[ti].at[:, pl.ds(j * _NG + step * _RB, _RB)]
        return es[ti].at[:, pl.ds(step * _RB, _RB)]

    def start_all(step, buf):
        for ti in range(4):
            for j in range(_G):
                pltpu.make_async_copy(
                    src(ti, j, step),
                    slab.at[buf, ti, pl.ds(j * _D, _D)], sem.at[buf]).start()

    @pl.when(i == 0)
    def _():
        start_all(0, 0)

    @pl.when(i + 1 < nblk)
    def _():
        start_all(i + 1, (i + 1) % 2)

    eye = jnp.eye(_GW, dtype=jnp.bfloat16)
    ys = (y1, y2, y3, y4)
    b = i % 2
    dims = (((0,), (0,)), ((), ()))
    for ti in range(4):
        for j in range(_G):
            pltpu.make_async_copy(
                src(ti, j, i), slab.at[b, ti, pl.ds(j * _D, _D)], sem.at[b]).wait()
        x = slab[b, ti]
        hi = x.astype(jnp.bfloat16)
        lo = (x - hi.astype(jnp.float32)).astype(jnp.bfloat16)
        yh = jax.lax.dot_general(hi, eye, dims,
                                 preferred_element_type=jnp.float32)
        yl = jax.lax.dot_general(lo, eye, dims,
                                 preferred_element_type=jnp.float32)
        ys[ti][...] = yh + yl


def _regroup(t1, t2, t3, t4):
    nblk = _NG // _RB
    in_spec = pl.BlockSpec(memory_space=pl.ANY)
    out_spec = pl.BlockSpec((_RB, _GW), lambda i: (i, 0))
    out = jax.ShapeDtypeStruct((_NG, _GW), jnp.float32)
    tails = [jnp.pad(t[:, _TAIL:], ((0, 0), (0, _NG - (_U - _TAIL))))
             for t in (t1, t2, t3, t4)]
    return pl.pallas_call(
        _regroup_body,
        grid=(nblk,),
        in_specs=[in_spec] * 8,
        out_specs=[out_spec] * 4,
        out_shape=[out] * 4,
        scratch_shapes=[
            pltpu.VMEM((2, 4, _GW, _RB), jnp.float32),
            pltpu.SemaphoreType.DMA((2,)),
        ],
    )(t1, t2, t3, t4, *tails)


def _extract(rows, j):
    acc = None
    for t in range(_G):
        part = jnp.where(j == t, rows[:, t * _D:(t + 1) * _D], 0.0)
        acc = part if acc is None else acc + part
    return acc


def _tower_body(gmfu, gmfp, gmfn, gmlu, gmlp, gmln, ju, jp, jn,
                w1u, w1i, b1, w2, b2, w3, b3, w4, b4, wdm, wdl, bd, out):
    f32 = jnp.float32
    ju_ = ju[...]
    jp_ = jp[...]
    jn_ = jn[...]
    mfu = _extract(gmfu[...], ju_)
    mlu = _extract(gmlu[...], ju_)
    mfp = _extract(gmfp[...], jp_)
    mlpos = _extract(gmlp[...], jp_)
    mfn = _extract(gmfn[...], jn_)
    mlneg = _extract(gmln[...], jn_)
    xu = jnp.dot(mlu, w1u[...], preferred_element_type=f32)
    hp = jnp.maximum(xu + jnp.dot(mlpos, w1i[...],
                                  preferred_element_type=f32) + b1[...], 0.0)
    hn = jnp.maximum(xu + jnp.dot(mlneg, w1i[...],
                                  preferred_element_type=f32) + b1[...], 0.0)
    for w, b in ((w2, b2), (w3, b3), (w4, b4)):
        hp = jnp.maximum(jnp.dot(hp, w[...], preferred_element_type=f32) + b[...], 0.0)
        hn = jnp.maximum(jnp.dot(hn, w[...], preferred_element_type=f32) + b[...], 0.0)
    mfp_v = jax.nn.sigmoid(mfu * mfp)
    mfn_v = jax.nn.sigmoid(mfu * mfn)
    sp = (jnp.dot(mfp_v, wdm[...], preferred_element_type=f32)
          + jnp.dot(hp, wdl[...], preferred_element_type=f32) + bd[0, 0])
    sn = (jnp.dot(mfn_v, wdm[...], preferred_element_type=f32)
          + jnp.dot(hn, wdl[...], preferred_element_type=f32) + bd[0, 0])
    out[...] = jnp.concatenate([sp, sn], axis=1)


def _tower(gmfu, gmfp, gmfn, gmlu, gmlp, gmln, ju, jp, jn,
           w1u, w1i, b1, w2, b2, w3, b3, w4, b4, wdm, wdl, bd,
           interpret=False):
    gspec = pl.BlockSpec((_BLK, _GW), lambda i: (i, 0))
    jspec = pl.BlockSpec((_BLK, 1), lambda i: (i, 0))

    def _full(a):
        return pl.BlockSpec(a.shape, lambda i: (0,) * a.ndim)

    weights = (w1u, w1i, b1, w2, b2, w3, b3, w4, b4, wdm, wdl, bd)
    return pl.pallas_call(
        _tower_body,
        grid=(_B // _BLK,),
        in_specs=[gspec] * 6 + [jspec] * 3 + [_full(w) for w in weights],
        out_specs=pl.BlockSpec((_BLK, 2), lambda i: (i, 0)),
        out_shape=jax.ShapeDtypeStruct((_B, 2), jnp.float32),
        interpret=interpret,
    )(gmfu, gmfp, gmfn, gmlu, gmlp, gmln, ju, jp, jn, *weights)


def kernel(user, pos_item, neg_item,
           mf_user_table, mf_item_table, mlp_user_table, mlp_item_table,
           W1, b1, W2, b2, W3, b3, W4, b4, Wd, bd):
    user = user.astype(jnp.int32)
    pos = pos_item.astype(jnp.int32)
    neg = neg_item.reshape(-1).astype(jnp.int32)
    g_mfu_t, g_mfi_t, g_mlu_t, g_mli_t = _regroup(
        mf_user_table.T, mf_item_table.T,
        mlp_user_table.T, mlp_item_table.T)
    gmfu, gmfp, gmfn, gmlu, gmlp, gmln = _gather6(
        user % _NG, pos % _NG, neg % _NG,
        g_mfu_t, g_mfi_t, g_mlu_t, g_mli_t)
    logits = _tower(
        gmfu, gmfp, gmfn, gmlu, gmlp, gmln,
        (user // _NG).astype(jnp.int32).reshape(-1, 1),
        (pos // _NG).astype(jnp.int32).reshape(-1, 1),
        (neg // _NG).astype(jnp.int32).reshape(-1, 1),
        W1[:_D], W1[_D:], b1.reshape(1, -1),
        W2, b2.reshape(1, -1), W3, b3.reshape(1, -1), W4, b4.reshape(1, -1),
        Wd[:_D], Wd[_D:], bd.reshape(1, 1))
    return logits


# speedup vs baseline: 3.7407x; 1.0206x over previous
import functools

import jax
import jax.numpy as jnp
from jax import lax
from jax.experimental import pallas as pl
from jax.experimental.pallas import tpu as pltpu
from jax.experimental.pallas import tpu_sc as plsc

_B = 16384
_D = 16
_G = 8
_GW = _G * _D
_NC = 2
_NS = 16
_NW = _NC * _NS
_BPW = _B // _NW
_CHUNK = 128
_NCHUNK = _BPW // _CHUNK

_BLK = 1024


def _gather_body(user_h, pos_h, neg_h, mfu_t, mfi_t, mlu_t, mli_t,
                 o_mfu, o_mfp, o_mfn, o_mlu, o_mlp, o_mln,
                 uidx, pidx, nidx, buf0, buf1,
                 sg0, sg1, so0, so1):
    wid = lax.axis_index("s") * _NC + lax.axis_index("c")
    base = wid * _BPW
    pltpu.sync_copy(user_h.at[pl.ds(base, _BPW)], uidx)
    pltpu.sync_copy(pos_h.at[pl.ds(base, _BPW)], pidx)
    pltpu.sync_copy(neg_h.at[pl.ds(base, _BPW)], nidx)

    units = []
    for tab, idx, dst in ((mfu_t, uidx, o_mfu), (mlu_t, uidx, o_mlu),
                          (mfi_t, pidx, o_mfp), (mli_t, pidx, o_mlp),
                          (mfi_t, nidx, o_mfn), (mli_t, nidx, o_mln)):
        for c in range(_NCHUNK):
            units.append((tab, idx, dst, c))

    bufs = (buf0, buf1)
    gsems = (sg0, sg1)
    osems = (so0, so1)

    def start_gather(u, b):
        tab, idx, _, c = units[u]
        return pltpu.async_copy(
            tab.at[idx.at[pl.ds(c * _CHUNK, _CHUNK)]], bufs[b], gsems[b])

    def start_out(u, b):
        _, _, dst, c = units[u]
        return pltpu.async_copy(
            bufs[b], dst.at[pl.ds(base + c * _CHUNK, _CHUNK)], osems[b])

    n = len(units)
    gath = start_gather(0, 0)
    outc = [None, None]
    for u in range(n):
        b = u % 2
        nb = 1 - b
        gath.wait()
        if u + 1 < n:
            if outc[nb] is not None:
                outc[nb].wait()
                outc[nb] = None
            next_gath = start_gather(u + 1, nb)
        outc[b] = start_out(u, b)
        if u + 1 < n:
            gath = next_gath
    for oc in outc:
        if oc is not None:
            oc.wait()


@jax.jit
def _gather6(user_g, pos_g, neg_g, mfu_t, mfi_t, mlu_t, mli_t):
    mesh = plsc.VectorSubcoreMesh(core_axis_name="c", subcore_axis_name="s")
    out = jax.ShapeDtypeStruct((_B, _GW), jnp.float32)
    f = pl.kernel(
        _gather_body,
        out_type=[out] * 6,
        mesh=mesh,
        scratch_types=(
            [pltpu.VMEM((_BPW,), jnp.int32)] * 3
            + [pltpu.VMEM((_CHUNK, _GW), jnp.float32)] * 2
            + [pltpu.SemaphoreType.DMA] * 4
        ),
    )
    return f(user_g, pos_g, neg_g, mfu_t, mfi_t, mlu_t, mli_t)


_U = 1000000
_NG = 125440
_RB = 4480
_TAIL = (_G - 1) * _NG


def _regroup_body(x1, x2, x3, x4, e1, e2, e3, e4, y1, y2, y3, y4, slab, sem):
    i = pl.program_id(0)
    nblk = pl.num_programs(0)
    xs = (x1, x2, x3, x4)
    es = (e1, e2, e3, e4)

    def src(ti, j, step):
        if j < _G - 1:
            return xs[ti].at[:, pl.ds(j * _NG + step * _RB, _RB)]
        return es[ti].at[:, pl.ds(step * _RB, _RB)]

    def start_all(step, buf):
        for ti in range(4):
            for j in range(_G):
                pltpu.make_async_copy(
                    src(ti, j, step),
                    slab.at[buf, ti, pl.ds(j * _D, _D)], sem.at[buf]).start()

    @pl.when(i == 0)
    def _():
        start_all(0, 0)

    @pl.when(i + 1 < nblk)
    def _():
        start_all(i + 1, (i + 1) % 2)

    eye = jnp.eye(_GW, dtype=jnp.bfloat16)
    ys = (y1, y2, y3, y4)
    b = i % 2
    dims = (((0,), (0,)), ((), ()))
    for ti in range(4):
        for j in range(_G):
            pltpu.make_async_copy(
                src(ti, j, i), slab.at[b, ti, pl.ds(j * _D, _D)], sem.at[b]).wait()
        x = slab[b, ti]
        hi = x.astype(jnp.bfloat16)
        lo = (x - hi.astype(jnp.float32)).astype(jnp.bfloat16)
        yh = jax.lax.dot_general(hi, eye, dims,
                                 preferred_element_type=jnp.float32)
        yl = jax.lax.dot_general(lo, eye, dims,
                                 preferred_element_type=jnp.float32)
        ys[ti][...] = yh + yl


def _regroup(t1, t2, t3, t4):
    nblk = _NG // _RB
    in_spec = pl.BlockSpec(memory_space=pl.ANY)
    out_spec = pl.BlockSpec((_RB, _GW), lambda i: (i, 0))
    out = jax.ShapeDtypeStruct((_NG, _GW), jnp.float32)
    tails = [jnp.pad(t[:, _TAIL:], ((0, 0), (0, _NG - (_U - _TAIL))))
             for t in (t1, t2, t3, t4)]
    return pl.pallas_call(
        _regroup_body,
        grid=(nblk,),
        in_specs=[in_spec] * 8,
        out_specs=[out_spec] * 4,
        out_shape=[out] * 4,
        scratch_shapes=[
            pltpu.VMEM((2, 4, _GW, _RB), jnp.float32),
            pltpu.SemaphoreType.DMA((2,)),
        ],
    )(t1, t2, t3, t4, *tails)


def _extract(rows, j):
    acc = None
    for t in range(_G):
        part = jnp.where(j == t, rows[:, t * _D:(t + 1) * _D], 0.0)
        acc = part if acc is None else acc + part
    return acc


def _tower_body(gmfu, gmfp, gmfn, gmlu, gmlp, gmln, ju, jp, jn,
                w1u, w1i, b1, w2, b2, w3, b3, w4, b4, wdm, wdl, bd, out):
    f32 = jnp.float32
    ju_ = ju[...] // _NG
    jp_ = jp[...] // _NG
    jn_ = jn[...] // _NG
    mfu = _extract(gmfu[...], ju_)
    mlu = _extract(gmlu[...], ju_)
    mfp = _extract(gmfp[...], jp_)
    mlpos = _extract(gmlp[...], jp_)
    mfn = _extract(gmfn[...], jn_)
    mlneg = _extract(gmln[...], jn_)
    xu = jnp.dot(mlu, w1u[...], preferred_element_type=f32)
    hp = jnp.maximum(xu + jnp.dot(mlpos, w1i[...],
                                  preferred_element_type=f32) + b1[...], 0.0)
    hn = jnp.maximum(xu + jnp.dot(mlneg, w1i[...],
                                  preferred_element_type=f32) + b1[...], 0.0)
    for w, b in ((w2, b2), (w3, b3), (w4, b4)):
        hp = jnp.maximum(jnp.dot(hp, w[...], preferred_element_type=f32) + b[...], 0.0)
        hn = jnp.maximum(jnp.dot(hn, w[...], preferred_element_type=f32) + b[...], 0.0)
    mfp_v = jax.nn.sigmoid(mfu * mfp)
    mfn_v = jax.nn.sigmoid(mfu * mfn)
    sp = (jnp.dot(mfp_v, wdm[...], preferred_element_type=f32)
          + jnp.dot(hp, wdl[...], preferred_element_type=f32) + bd[0, 0])
    sn = (jnp.dot(mfn_v, wdm[...], preferred_element_type=f32)
          + jnp.dot(hn, wdl[...], preferred_element_type=f32) + bd[0, 0])
    out[...] = jnp.concatenate([sp, sn], axis=1)


def _tower(gmfu, gmfp, gmfn, gmlu, gmlp, gmln, ju, jp, jn,
           w1u, w1i, b1, w2, b2, w3, b3, w4, b4, wdm, wdl, bd,
           interpret=False):
    gspec = pl.BlockSpec((_BLK, _GW), lambda i: (i, 0))
    jspec = pl.BlockSpec((_BLK, 1), lambda i: (i, 0))

    def _full(a):
        return pl.BlockSpec(a.shape, lambda i: (0,) * a.ndim)

    weights = (w1u, w1i, b1, w2, b2, w3, b3, w4, b4, wdm, wdl, bd)
    return pl.pallas_call(
        _tower_body,
        grid=(_B // _BLK,),
        in_specs=[gspec] * 6 + [jspec] * 3 + [_full(w) for w in weights],
        out_specs=pl.BlockSpec((_BLK, 2), lambda i: (i, 0)),
        out_shape=jax.ShapeDtypeStruct((_B, 2), jnp.float32),
        interpret=interpret,
    )(gmfu, gmfp, gmfn, gmlu, gmlp, gmln, ju, jp, jn, *weights)


def kernel(user, pos_item, neg_item,
           mf_user_table, mf_item_table, mlp_user_table, mlp_item_table,
           W1, b1, W2, b2, W3, b3, W4, b4, Wd, bd):
    user = user.astype(jnp.int32)
    pos = pos_item.astype(jnp.int32)
    neg = neg_item.reshape(-1).astype(jnp.int32)
    g_mfu_t, g_mfi_t, g_mlu_t, g_mli_t = _regroup(
        mf_user_table.T, mf_item_table.T,
        mlp_user_table.T, mlp_item_table.T)
    gmfu, gmfp, gmfn, gmlu, gmlp, gmln = _gather6(
        user % _NG, pos % _NG, neg % _NG,
        g_mfu_t, g_mfi_t, g_mlu_t, g_mli_t)
    logits = _tower(
        gmfu, gmfp, gmfn, gmlu, gmlp, gmln,
        user.reshape(-1, 1), pos.reshape(-1, 1), neg.reshape(-1, 1),
        W1[:_D], W1[_D:], b1.reshape(1, -1),
        W2, b2.reshape(1, -1), W3, b3.reshape(1, -1), W4, b4.reshape(1, -1),
        Wd[:_D], Wd[_D:], bd.reshape(1, 1))
    return logits
